# speedup vs baseline: 3.6004x; 3.6004x over previous
import functools

import jax
import jax.numpy as jnp
from jax import lax
from jax.experimental import pallas as pl
from jax.experimental.pallas import tpu as pltpu
from jax.experimental.pallas import tpu_sc as plsc

_B = 4
_S = 4096
_D = 128
_N = _B * _S


@functools.cache
def _gather_kernel():
    info = plsc.get_sparse_core_info()
    nw = info.num_cores * info.num_subcores
    per_w = _N // nw
    ch = 256
    n_ch = per_w // ch
    mesh = plsc.VectorSubcoreMesh(core_axis_name="c", subcore_axis_name="s")

    @functools.partial(
        pl.kernel,
        mesh=mesh,
        out_type=[
            jax.ShapeDtypeStruct((_N, _D), jnp.float32),
            jax.ShapeDtypeStruct((_N, _D), jnp.float32),
        ],
        scratch_types=[
            pltpu.VMEM((per_w,), jnp.int32),
            pltpu.VMEM((ch, _D), jnp.float32),
            pltpu.VMEM((ch, _D), jnp.float32),
            pltpu.SemaphoreType.DMA,
            pltpu.SemaphoreType.DMA,
        ],
    )
    def k(cos_hbm, sin_hbm, idx_hbm, cos_out, sin_out,
          idx_v, buf_c, buf_s, sem_c, sem_s):
        wid = lax.axis_index("s") * info.num_cores + lax.axis_index("c")
        base = wid * per_w
        pltpu.sync_copy(idx_hbm.at[pl.ds(base, per_w)], idx_v)
        for c in range(n_ch):
            idx_sl = idx_v.at[pl.ds(c * ch, ch)]
            cpy_c = pltpu.make_async_copy(cos_hbm.at[idx_sl], buf_c, sem_c)
            cpy_s = pltpu.make_async_copy(sin_hbm.at[idx_sl], buf_s, sem_s)
            cpy_c.start()
            cpy_s.start()
            out_sl = pl.ds(base + c * ch, ch)
            cpy_c.wait()
            pltpu.sync_copy(buf_c, cos_out.at[out_sl])
            cpy_s.wait()
            pltpu.sync_copy(buf_s, sin_out.at[out_sl])

    return k


@jax.jit
def kernel(x, position_ids, cos_cached, sin_cached):
    idx = position_ids.reshape(_N).astype(jnp.int32)
    cos_flat, sin_flat = _gather_kernel()(cos_cached, sin_cached, idx)
    cos = cos_flat.reshape(_B, 1, _S, _D)
    sin = sin_flat.reshape(_B, 1, _S, _D)
    return (cos, sin)


# speedup vs baseline: 3.7487x; 1.0412x over previous
import functools

import jax
import jax.numpy as jnp
from jax import lax
from jax.experimental import pallas as pl
from jax.experimental.pallas import tpu as pltpu
from jax.experimental.pallas import tpu_sc as plsc

_B = 4
_S = 4096
_D = 128
_N = _B * _S


@functools.cache
def _gather_kernel():
    info = plsc.get_sparse_core_info()
    nw = info.num_cores * info.num_subcores
    per_w = _N // nw
    ch = 128
    n_ch = per_w // ch
    nbuf = 4
    ntask = 2 * n_ch
    mesh = plsc.VectorSubcoreMesh(core_axis_name="c", subcore_axis_name="s")

    @functools.partial(
        pl.kernel,
        mesh=mesh,
        out_type=[
            jax.ShapeDtypeStruct((_N, _D), jnp.float32),
            jax.ShapeDtypeStruct((_N, _D), jnp.float32),
        ],
        scratch_types=[
            pltpu.VMEM((per_w,), jnp.int32),
        ]
        + [pltpu.VMEM((ch, _D), jnp.float32) for _ in range(nbuf)]
        + [pltpu.SemaphoreType.DMA for _ in range(2 * nbuf)],
    )
    def k(cos_hbm, sin_hbm, idx_hbm, cos_out, sin_out, idx_v, *bufs_sems):
        bufs = bufs_sems[:nbuf]
        gsem = bufs_sems[nbuf:2 * nbuf]
        wsem = bufs_sems[2 * nbuf:]
        wid = lax.axis_index("s") * info.num_cores + lax.axis_index("c")
        base = wid * per_w

        def task(t):
            c = t // 2
            tab = cos_hbm if t % 2 == 0 else sin_hbm
            out = cos_out if t % 2 == 0 else sin_out
            return tab, out, pl.ds(c * ch, ch), pl.ds(base + c * ch, ch)

        pltpu.sync_copy(idx_hbm.at[pl.ds(base, per_w)], idx_v)
        for t in range(nbuf):
            tab, _, in_sl, _ = task(t)
            pltpu.make_async_copy(tab.at[idx_v.at[in_sl]], bufs[t], gsem[t]).start()
        for t in range(ntask):
            b = t % nbuf
            tab, out, in_sl, out_sl = task(t)
            pltpu.make_async_copy(tab.at[idx_v.at[in_sl]], bufs[b], gsem[b]).wait()
            wb = pltpu.make_async_copy(bufs[b], out.at[out_sl], wsem[b])
            wb.start()
            nt = t + nbuf
            if nt < ntask:
                wb.wait()
                ntab, _, nin_sl, _ = task(nt)
                pltpu.make_async_copy(ntab.at[idx_v.at[nin_sl]], bufs[b], gsem[b]).start()
            else:
                wb.wait()

    return k


@jax.jit
def kernel(x, position_ids, cos_cached, sin_cached):
    idx = position_ids.reshape(_N).astype(jnp.int32)
    cos_flat, sin_flat = _gather_kernel()(cos_cached, sin_cached, idx)
    cos = cos_flat.reshape(_B, 1, _S, _D)
    sin = sin_flat.reshape(_B, 1, _S, _D)
    return (cos, sin)
